# baseline (speedup 1.0000x reference)
import functools

import jax
import jax.numpy as jnp
from jax.experimental import pallas as pl
from jax.experimental.pallas import tpu as pltpu

N_TOK_ = 4096
D_MODEL_ = 2048
D_SAE_ = 16384
K_ = 64

MINI32 = -2147483648


def _sortable(z):
    b = jax.lax.bitcast_convert_type(z, jnp.int32)
    return jnp.where(b < 0, jnp.bitwise_xor(~b, jnp.int32(MINI32)), b)


def _encode_kernel(x_ref, bpre_ref, w_ref, benc_ref, z_ref):
    xb = x_ref[...] - bpre_ref[...]
    zb = jax.lax.dot_general(
        xb.astype(jnp.bfloat16), w_ref[...].astype(jnp.bfloat16),
        dimension_numbers=(((1,), (1,)), ((), ())),
        preferred_element_type=jnp.float32,
    )
    z_ref[...] = zb + benc_ref[...]


def _select_kernel(z_ref, tau_ref, *, k):
    s = _sortable(z_ref[...])
    hi_u = jnp.bitwise_xor(jax.lax.shift_right_logical(s, 16), 0x8000)
    s16 = (hi_u - 32768).astype(jnp.int16)
    ones16 = jnp.ones((s.shape[1], 8), jnp.bfloat16)
    ones32 = jnp.ones((s.shape[1], 8), jnp.float32)

    def count_ge16(mask):
        mb = jnp.where(mask, jnp.bfloat16(1), jnp.bfloat16(0))
        c = jax.lax.dot_general(mb, ones16, (((1,), (0,)), ((), ())),
                                preferred_element_type=jnp.float32)
        return c[:, :1]

    def count_ge32(mask):
        mb = jnp.where(mask, jnp.float32(1), jnp.float32(0))
        c = jax.lax.dot_general(mb, ones32, (((1,), (0,)), ((), ())),
                                preferred_element_type=jnp.float32)
        return c[:, :1]

    def body_hi(i, p):
        bit = jax.lax.shift_left(jnp.int32(1), 15 - i)
        cand = jnp.bitwise_or(p, bit)
        cand16 = (cand - 32768).astype(jnp.int16)
        cnt = count_ge16(s16 >= cand16)
        return jnp.where(cnt >= k, cand, p)

    p_hi = jax.lax.fori_loop(0, 16, body_hi,
                             jnp.zeros(tau_ref.shape, jnp.int32))
    base = jax.lax.shift_left(p_hi, 16)

    def body_lo(i, p):
        bit = jax.lax.shift_left(jnp.int32(1), 15 - i)
        cand = jnp.bitwise_or(p, bit)
        candb = jnp.bitwise_xor(cand, jnp.int32(MINI32))
        cnt = count_ge32(s >= candb)
        return jnp.where(cnt >= k, cand, p)

    p = jax.lax.fori_loop(0, 16, body_lo, base)
    tau_ref[...] = jnp.bitwise_xor(p, jnp.int32(MINI32))


def _decode_kernel(z_ref, tau_ref, wd_ref, bdec_ref, zs_ref, rec_ref):
    zb = z_ref[...]
    mask = _sortable(zb) >= tau_ref[...]
    zs = jnp.where(mask, zb, 0.0)
    zs_ref[...] = zs

    contrib = jax.lax.dot_general(
        zs.astype(jnp.bfloat16), wd_ref[...].astype(jnp.bfloat16),
        dimension_numbers=(((1,), (1,)), ((), ())),
        preferred_element_type=jnp.float32,
    )

    @pl.when(pl.program_id(1) == 0)
    def _init():
        rec_ref[...] = jnp.broadcast_to(bdec_ref[...], rec_ref.shape)

    rec_ref[...] += contrib


@jax.jit
def kernel(x, b_pre, W_enc, b_enc, W_dec, b_dec):
    n_tok, d_model = x.shape
    d_sae = W_enc.shape[0]

    TB_E, FB_E = 512, 1024
    z = pl.pallas_call(
        _encode_kernel,
        grid=(d_sae // FB_E, n_tok // TB_E),
        in_specs=[
            pl.BlockSpec((TB_E, d_model), lambda f, t: (t, 0)),
            pl.BlockSpec((1, d_model), lambda f, t: (0, 0)),
            pl.BlockSpec((FB_E, d_model), lambda f, t: (f, 0)),
            pl.BlockSpec((1, FB_E), lambda f, t: (0, f)),
        ],
        out_specs=pl.BlockSpec((TB_E, FB_E), lambda f, t: (t, f)),
        out_shape=jax.ShapeDtypeStruct((n_tok, d_sae), jnp.float32),
        compiler_params=pltpu.CompilerParams(
            dimension_semantics=("arbitrary", "arbitrary")),
    )(x, b_pre.reshape(1, d_model), W_enc, b_enc.reshape(1, d_sae))

    TB_S = 128
    tau = pl.pallas_call(
        functools.partial(_select_kernel, k=K_),
        grid=(n_tok // TB_S,),
        in_specs=[pl.BlockSpec((TB_S, d_sae), lambda t: (t, 0))],
        out_specs=pl.BlockSpec((TB_S, 1), lambda t: (t, 0)),
        out_shape=jax.ShapeDtypeStruct((n_tok, 1), jnp.int32),
        compiler_params=pltpu.CompilerParams(
            dimension_semantics=("arbitrary",)),
    )(z)

    TB_D, KB_D = 1024, 512
    z_sparse, recon = pl.pallas_call(
        _decode_kernel,
        grid=(n_tok // TB_D, d_sae // KB_D),
        in_specs=[
            pl.BlockSpec((TB_D, KB_D), lambda t, kk: (t, kk)),
            pl.BlockSpec((TB_D, 1), lambda t, kk: (t, 0)),
            pl.BlockSpec((d_model, KB_D), lambda t, kk: (0, kk)),
            pl.BlockSpec((1, d_model), lambda t, kk: (0, 0)),
        ],
        out_specs=[
            pl.BlockSpec((TB_D, KB_D), lambda t, kk: (t, kk)),
            pl.BlockSpec((TB_D, d_model), lambda t, kk: (t, 0)),
        ],
        out_shape=[
            jax.ShapeDtypeStruct((n_tok, d_sae), jnp.float32),
            jax.ShapeDtypeStruct((n_tok, d_model), jnp.float32),
        ],
        compiler_params=pltpu.CompilerParams(
            dimension_semantics=("arbitrary", "arbitrary")),
    )(z, tau, W_dec, b_dec.reshape(1, d_model))

    return (recon, z_sparse)


# speedup vs baseline: 1.2032x; 1.2032x over previous
import functools

import jax
import jax.numpy as jnp
from jax.experimental import pallas as pl
from jax.experimental.pallas import tpu as pltpu

N_TOK_ = 4096
D_MODEL_ = 2048
D_SAE_ = 16384
K_ = 64

MINI32 = -2147483648


def _sortable(z):
    b = jax.lax.bitcast_convert_type(z, jnp.int32)
    return jnp.where(b < 0, jnp.bitwise_xor(~b, jnp.int32(MINI32)), b)


def _encode_kernel(x_ref, bpre_ref, w_ref, benc_ref, z_ref):
    xb = x_ref[...] - bpre_ref[...]
    zb = jax.lax.dot_general(
        xb.astype(jnp.bfloat16), w_ref[...].astype(jnp.bfloat16),
        dimension_numbers=(((1,), (1,)), ((), ())),
        preferred_element_type=jnp.float32,
    )
    z_ref[...] = zb + benc_ref[...]


def _select_kernel(z_ref, tau_ref, *, k):
    s = _sortable(z_ref[...])
    n = s.shape[1]
    hi_u = jnp.bitwise_xor(jax.lax.shift_right_logical(s, 16), 0x8000)
    s16 = (hi_u - 32768).astype(jnp.int16)
    lo_u = jnp.bitwise_and(s, 0xFFFF)
    lo16 = (lo_u - 32768).astype(jnp.int16)

    def count_ge(mask):
        ones = jnp.where(mask, jnp.int16(1), jnp.int16(0))
        acc = ones[:, : n // 16]
        for j in range(1, 16):
            acc = acc + ones[:, j * (n // 16):(j + 1) * (n // 16)]
        acc2 = acc[:, : n // 256]
        for j in range(1, 16):
            acc2 = acc2 + acc[:, j * (n // 256):(j + 1) * (n // 256)]
        return jnp.sum(acc2.astype(jnp.int32), axis=1, keepdims=True)

    def body_hi(i, p):
        bit = jax.lax.shift_left(jnp.int32(1), 15 - i)
        cand = jnp.bitwise_or(p, bit)
        cand16 = (cand - 32768).astype(jnp.int16)
        cnt = count_ge(s16 >= cand16)
        return jnp.where(cnt >= k, cand, p)

    p_hi = jax.lax.fori_loop(0, 16, body_hi,
                             jnp.zeros(tau_ref.shape, jnp.int32))

    p16 = (p_hi - 32768).astype(jnp.int16)
    above = count_ge(s16 >= ((p_hi + 1) - 32768).astype(jnp.int16))
    l16m = jnp.where(s16 == p16, lo16, jnp.int16(-32768))

    def body_lo(i, p):
        bit = jax.lax.shift_left(jnp.int32(1), 15 - i)
        cand = jnp.bitwise_or(p, bit)
        cand16 = (cand - 32768).astype(jnp.int16)
        cnt = above + count_ge(l16m >= cand16)
        return jnp.where(cnt >= k, cand, p)

    p_lo = jax.lax.fori_loop(0, 16, body_lo,
                             jnp.zeros(tau_ref.shape, jnp.int32))
    p = jnp.bitwise_or(jax.lax.shift_left(p_hi, 16), p_lo)
    tau_ref[...] = jnp.bitwise_xor(p, jnp.int32(MINI32))


def _decode_kernel(z_ref, tau_ref, wd_ref, bdec_ref, zs_ref, rec_ref):
    zb = z_ref[...]
    mask = _sortable(zb) >= tau_ref[...]
    zs = jnp.where(mask, zb, 0.0)
    zs_ref[...] = zs

    contrib = jax.lax.dot_general(
        zs.astype(jnp.bfloat16), wd_ref[...].astype(jnp.bfloat16),
        dimension_numbers=(((1,), (1,)), ((), ())),
        preferred_element_type=jnp.float32,
    )

    @pl.when(pl.program_id(1) == 0)
    def _init():
        rec_ref[...] = jnp.broadcast_to(bdec_ref[...], rec_ref.shape)

    rec_ref[...] += contrib


@jax.jit
def kernel(x, b_pre, W_enc, b_enc, W_dec, b_dec):
    n_tok, d_model = x.shape
    d_sae = W_enc.shape[0]

    TB_E, FB_E = 512, 1024
    z = pl.pallas_call(
        _encode_kernel,
        grid=(d_sae // FB_E, n_tok // TB_E),
        in_specs=[
            pl.BlockSpec((TB_E, d_model), lambda f, t: (t, 0)),
            pl.BlockSpec((1, d_model), lambda f, t: (0, 0)),
            pl.BlockSpec((FB_E, d_model), lambda f, t: (f, 0)),
            pl.BlockSpec((1, FB_E), lambda f, t: (0, f)),
        ],
        out_specs=pl.BlockSpec((TB_E, FB_E), lambda f, t: (t, f)),
        out_shape=jax.ShapeDtypeStruct((n_tok, d_sae), jnp.float32),
        compiler_params=pltpu.CompilerParams(
            dimension_semantics=("arbitrary", "arbitrary")),
    )(x, b_pre.reshape(1, d_model), W_enc, b_enc.reshape(1, d_sae))

    TB_S = 128
    tau = pl.pallas_call(
        functools.partial(_select_kernel, k=K_),
        grid=(n_tok // TB_S,),
        in_specs=[pl.BlockSpec((TB_S, d_sae), lambda t: (t, 0))],
        out_specs=pl.BlockSpec((TB_S, 1), lambda t: (t, 0)),
        out_shape=jax.ShapeDtypeStruct((n_tok, 1), jnp.int32),
        compiler_params=pltpu.CompilerParams(
            dimension_semantics=("arbitrary",)),
    )(z)

    TB_D, KB_D = 512, 512
    z_sparse, recon = pl.pallas_call(
        _decode_kernel,
        grid=(n_tok // TB_D, d_sae // KB_D),
        in_specs=[
            pl.BlockSpec((TB_D, KB_D), lambda t, kk: (t, kk)),
            pl.BlockSpec((TB_D, 1), lambda t, kk: (t, 0)),
            pl.BlockSpec((d_model, KB_D), lambda t, kk: (0, kk)),
            pl.BlockSpec((1, d_model), lambda t, kk: (0, 0)),
        ],
        out_specs=[
            pl.BlockSpec((TB_D, KB_D), lambda t, kk: (t, kk)),
            pl.BlockSpec((TB_D, d_model), lambda t, kk: (t, 0)),
        ],
        out_shape=[
            jax.ShapeDtypeStruct((n_tok, d_sae), jnp.float32),
            jax.ShapeDtypeStruct((n_tok, d_model), jnp.float32),
        ],
        compiler_params=pltpu.CompilerParams(
            dimension_semantics=("arbitrary", "arbitrary")),
    )(z, tau, W_dec, b_dec.reshape(1, d_model))

    return (recon, z_sparse)


# speedup vs baseline: 1.4447x; 1.2008x over previous
import functools

import jax
import jax.numpy as jnp
from jax.experimental import pallas as pl
from jax.experimental.pallas import tpu as pltpu

N_TOK_ = 4096
D_MODEL_ = 2048
D_SAE_ = 16384
K_ = 64

MINI32 = -2147483648


def _sortable(z):
    b = jax.lax.bitcast_convert_type(z, jnp.int32)
    return jnp.where(b < 0, jnp.bitwise_xor(~b, jnp.int32(MINI32)), b)


def _encode_kernel(x_ref, bpre_ref, w_ref, benc_ref, z_ref):
    xb = x_ref[...] - bpre_ref[...]
    zb = jax.lax.dot_general(
        xb.astype(jnp.bfloat16), w_ref[...].astype(jnp.bfloat16),
        dimension_numbers=(((1,), (1,)), ((), ())),
        preferred_element_type=jnp.float32,
    )
    z_ref[...] = zb + benc_ref[...]


def _select_kernel(z_ref, tau_ref, *, k):
    s = _sortable(z_ref[...])
    n = s.shape[1]
    hi_u = jnp.bitwise_xor(jax.lax.shift_right_logical(s, 16), 0x8000)
    s16 = (hi_u - 32768).astype(jnp.int16)
    lo_u = jnp.bitwise_and(s, 0xFFFF)
    lo16 = (lo_u - 32768).astype(jnp.int16)

    def count_ge(mask):
        ones = jnp.where(mask, jnp.int16(1), jnp.int16(0))
        acc = ones[:, : n // 16]
        for j in range(1, 16):
            acc = acc + ones[:, j * (n // 16):(j + 1) * (n // 16)]
        return jnp.sum(acc.astype(jnp.int32), axis=1, keepdims=True)

    def body_hi(i, p):
        bit = jax.lax.shift_left(jnp.int32(1), 15 - i)
        cand = jnp.bitwise_or(p, bit)
        cand16 = (cand - 32768).astype(jnp.int16)
        cnt = count_ge(s16 >= cand16)
        return jnp.where(cnt >= k, cand, p)

    p_hi = jax.lax.fori_loop(0, 16, body_hi,
                             jnp.zeros(tau_ref.shape, jnp.int32))

    p16 = (p_hi - 32768).astype(jnp.int16)
    above = count_ge(s16 >= ((p_hi + 1) - 32768).astype(jnp.int16))
    l16m = jnp.where(s16 == p16, lo16, jnp.int16(-32768))

    def body_lo(i, p):
        bit = jax.lax.shift_left(jnp.int32(1), 15 - i)
        cand = jnp.bitwise_or(p, bit)
        cand16 = (cand - 32768).astype(jnp.int16)
        cnt = above + count_ge(l16m >= cand16)
        return jnp.where(cnt >= k, cand, p)

    p_lo = jax.lax.fori_loop(0, 16, body_lo,
                             jnp.zeros(tau_ref.shape, jnp.int32))
    p = jnp.bitwise_or(jax.lax.shift_left(p_hi, 16), p_lo)
    tau_ref[...] = jnp.bitwise_xor(p, jnp.int32(MINI32))


def _decode_kernel(z_ref, tau_ref, wd_ref, bdec_ref, zs_ref, rec_ref):
    zb = z_ref[...]
    mask = _sortable(zb) >= tau_ref[...]
    zs = jnp.where(mask, zb, 0.0)
    zs_ref[...] = zs

    contrib = jax.lax.dot_general(
        zs.astype(jnp.bfloat16), wd_ref[...].astype(jnp.bfloat16),
        dimension_numbers=(((1,), (1,)), ((), ())),
        preferred_element_type=jnp.float32,
    )

    @pl.when(pl.program_id(1) == 0)
    def _init():
        rec_ref[...] = jnp.broadcast_to(bdec_ref[...], rec_ref.shape)

    rec_ref[...] += contrib


@jax.jit
def kernel(x, b_pre, W_enc, b_enc, W_dec, b_dec):
    n_tok, d_model = x.shape
    d_sae = W_enc.shape[0]

    TB_E, FB_E = 512, 2048
    z = pl.pallas_call(
        _encode_kernel,
        grid=(d_sae // FB_E, n_tok // TB_E),
        in_specs=[
            pl.BlockSpec((TB_E, d_model), lambda f, t: (t, 0)),
            pl.BlockSpec((1, d_model), lambda f, t: (0, 0)),
            pl.BlockSpec((FB_E, d_model), lambda f, t: (f, 0)),
            pl.BlockSpec((1, FB_E), lambda f, t: (0, f)),
        ],
        out_specs=pl.BlockSpec((TB_E, FB_E), lambda f, t: (t, f)),
        out_shape=jax.ShapeDtypeStruct((n_tok, d_sae), jnp.float32),
        compiler_params=pltpu.CompilerParams(
            dimension_semantics=("arbitrary", "arbitrary")),
    )(x, b_pre.reshape(1, d_model), W_enc, b_enc.reshape(1, d_sae))

    TB_S = 128
    tau = pl.pallas_call(
        functools.partial(_select_kernel, k=K_),
        grid=(n_tok // TB_S,),
        in_specs=[pl.BlockSpec((TB_S, d_sae), lambda t: (t, 0))],
        out_specs=pl.BlockSpec((TB_S, 1), lambda t: (t, 0)),
        out_shape=jax.ShapeDtypeStruct((n_tok, 1), jnp.int32),
        compiler_params=pltpu.CompilerParams(
            dimension_semantics=("arbitrary",)),
    )(z)

    TB_D, KB_D = 1024, 512
    z_sparse, recon = pl.pallas_call(
        _decode_kernel,
        grid=(n_tok // TB_D, d_sae // KB_D),
        in_specs=[
            pl.BlockSpec((TB_D, KB_D), lambda t, kk: (t, kk)),
            pl.BlockSpec((TB_D, 1), lambda t, kk: (t, 0)),
            pl.BlockSpec((d_model, KB_D), lambda t, kk: (0, kk)),
            pl.BlockSpec((1, d_model), lambda t, kk: (0, 0)),
        ],
        out_specs=[
            pl.BlockSpec((TB_D, KB_D), lambda t, kk: (t, kk)),
            pl.BlockSpec((TB_D, d_model), lambda t, kk: (t, 0)),
        ],
        out_shape=[
            jax.ShapeDtypeStruct((n_tok, d_sae), jnp.float32),
            jax.ShapeDtypeStruct((n_tok, d_model), jnp.float32),
        ],
        compiler_params=pltpu.CompilerParams(
            dimension_semantics=("arbitrary", "arbitrary")),
    )(z, tau, W_dec, b_dec.reshape(1, d_model))

    return (recon, z_sparse)


# speedup vs baseline: 2.0662x; 1.4302x over previous
import functools

import jax
import jax.numpy as jnp
from jax.experimental import pallas as pl
from jax.experimental.pallas import tpu as pltpu
from jax.experimental.shard_map import shard_map
from jax.sharding import Mesh, PartitionSpec as P

K_ = 64

MINI32 = -2147483648


def _sortable(z):
    b = jax.lax.bitcast_convert_type(z, jnp.int32)
    return jnp.where(b < 0, jnp.bitwise_xor(~b, jnp.int32(MINI32)), b)


def _encode_kernel(x_ref, w_ref, benc_ref, z_ref):
    zb = jax.lax.dot_general(
        x_ref[...], w_ref[...],
        dimension_numbers=(((1,), (1,)), ((), ())),
        preferred_element_type=jnp.float32,
    )
    z_ref[...] = zb + benc_ref[...]


def _select_kernel(z_ref, tau_ref, *, k):
    s = _sortable(z_ref[...])
    n = s.shape[1]
    hi_u = jnp.bitwise_xor(jax.lax.shift_right_logical(s, 16), 0x8000)
    s16 = (hi_u - 32768).astype(jnp.int16)
    lo_u = jnp.bitwise_and(s, 0xFFFF)
    lo16 = (lo_u - 32768).astype(jnp.int16)

    def count_ge(mask):
        ones = jnp.where(mask, jnp.int16(1), jnp.int16(0))
        acc = ones[:, : n // 16]
        for j in range(1, 16):
            acc = acc + ones[:, j * (n // 16):(j + 1) * (n // 16)]
        return jnp.sum(acc.astype(jnp.int32), axis=1, keepdims=True)

    def body_hi(i, p):
        bit = jax.lax.shift_left(jnp.int32(1), 15 - i)
        cand = jnp.bitwise_or(p, bit)
        cand16 = (cand - 32768).astype(jnp.int16)
        cnt = count_ge(s16 >= cand16)
        return jnp.where(cnt >= k, cand, p)

    p_hi = jax.lax.fori_loop(0, 16, body_hi,
                             jnp.zeros(tau_ref.shape, jnp.int32))

    p16 = (p_hi - 32768).astype(jnp.int16)
    above = count_ge(s16 >= ((p_hi + 1) - 32768).astype(jnp.int16))
    l16m = jnp.where(s16 == p16, lo16, jnp.int16(-32768))

    def body_lo(i, p):
        bit = jax.lax.shift_left(jnp.int32(1), 15 - i)
        cand = jnp.bitwise_or(p, bit)
        cand16 = (cand - 32768).astype(jnp.int16)
        cnt = above + count_ge(l16m >= cand16)
        return jnp.where(cnt >= k, cand, p)

    p_lo = jax.lax.fori_loop(0, 16, body_lo,
                             jnp.zeros(tau_ref.shape, jnp.int32))
    p = jnp.bitwise_or(jax.lax.shift_left(p_hi, 16), p_lo)
    tau_ref[...] = jnp.bitwise_xor(p, jnp.int32(MINI32))


def _decode_kernel(z_ref, tau_ref, wd_ref, bdec_ref, zs_ref, rec_ref):
    zb = z_ref[...]
    mask = _sortable(zb) >= tau_ref[...]
    zs = jnp.where(mask, zb, 0.0)
    zs_ref[...] = zs

    contrib = jax.lax.dot_general(
        zs.astype(jnp.bfloat16), wd_ref[...],
        dimension_numbers=(((1,), (1,)), ((), ())),
        preferred_element_type=jnp.float32,
    )

    @pl.when(pl.program_id(1) == 0)
    def _init():
        rec_ref[...] = jnp.broadcast_to(bdec_ref[...], rec_ref.shape)

    rec_ref[...] += contrib


def _pipeline(xb, we, b_enc, wd, b_dec):
    n_tok, d_model = xb.shape
    d_sae = we.shape[0]

    TB_E, FB_E = 512, 2048
    z = pl.pallas_call(
        _encode_kernel,
        grid=(d_sae // FB_E, n_tok // TB_E),
        in_specs=[
            pl.BlockSpec((TB_E, d_model), lambda f, t: (t, 0)),
            pl.BlockSpec((FB_E, d_model), lambda f, t: (f, 0)),
            pl.BlockSpec((1, FB_E), lambda f, t: (0, f)),
        ],
        out_specs=pl.BlockSpec((TB_E, FB_E), lambda f, t: (t, f)),
        out_shape=jax.ShapeDtypeStruct((n_tok, d_sae), jnp.float32),
        compiler_params=pltpu.CompilerParams(
            dimension_semantics=("arbitrary", "arbitrary")),
    )(xb, we, b_enc.reshape(1, d_sae))

    TB_S = 128
    tau = pl.pallas_call(
        functools.partial(_select_kernel, k=K_),
        grid=(n_tok // TB_S,),
        in_specs=[pl.BlockSpec((TB_S, d_sae), lambda t: (t, 0))],
        out_specs=pl.BlockSpec((TB_S, 1), lambda t: (t, 0)),
        out_shape=jax.ShapeDtypeStruct((n_tok, 1), jnp.int32),
        compiler_params=pltpu.CompilerParams(
            dimension_semantics=("arbitrary",)),
    )(z)

    TB_D, KB_D = 1024, 512
    z_sparse, recon = pl.pallas_call(
        _decode_kernel,
        grid=(n_tok // TB_D, d_sae // KB_D),
        in_specs=[
            pl.BlockSpec((TB_D, KB_D), lambda t, kk: (t, kk)),
            pl.BlockSpec((TB_D, 1), lambda t, kk: (t, 0)),
            pl.BlockSpec((d_model, KB_D), lambda t, kk: (0, kk)),
            pl.BlockSpec((1, d_model), lambda t, kk: (0, 0)),
        ],
        out_specs=[
            pl.BlockSpec((TB_D, KB_D), lambda t, kk: (t, kk)),
            pl.BlockSpec((TB_D, d_model), lambda t, kk: (t, 0)),
        ],
        out_shape=[
            jax.ShapeDtypeStruct((n_tok, d_sae), jnp.float32),
            jax.ShapeDtypeStruct((n_tok, d_model), jnp.float32),
        ],
        compiler_params=pltpu.CompilerParams(
            dimension_semantics=("arbitrary", "arbitrary")),
    )(z, tau, wd, b_dec.reshape(1, d_model))

    return (recon, z_sparse)


@jax.jit
def kernel(x, b_pre, W_enc, b_enc, W_dec, b_dec):
    xb = (x - b_pre).astype(jnp.bfloat16)
    we = W_enc.astype(jnp.bfloat16)
    wd = W_dec.astype(jnp.bfloat16)

    devs = jax.devices()
    n_tok = x.shape[0]
    nd = 2 if (len(devs) >= 2 and n_tok % 2 == 0) else 1
    if nd == 1:
        return _pipeline(xb, we, b_enc, wd, b_dec)

    mesh = Mesh(devs[:nd], ("t",))
    f = shard_map(
        _pipeline, mesh=mesh,
        in_specs=(P("t", None), P(None, None), P(None), P(None, None),
                  P(None)),
        out_specs=(P("t", None), P("t", None)),
        check_rep=False,
    )
    return f(xb, we, b_enc, wd, b_dec)
